# baseline (speedup 1.0000x reference)
import functools

import jax
import jax.numpy as jnp
from jax import lax
from jax.experimental import pallas as pl
from jax.experimental.pallas import tpu as pltpu
from jax.experimental.pallas import tpu_sc as plsc

H = 12
HD = 64
D = 768
E = 64
CAP = 80
TOPK = 2
N_TOK = 2048
P_PAIRS = N_TOK * TOPK
NW = 32
PPW = P_PAIRS // NW
TRASH = E * CAP


def _f32(x):
    return jnp.asarray(x, jnp.float32)


def _mods_kernel(t_ref, w_ref, b_ref, o_ref):
    s = jax.nn.silu(t_ref[...])
    o_ref[...] = jnp.dot(s, w_ref[...], preferred_element_type=jnp.float32) + b_ref[...]


def _mods(t2d, wcat, bcat):
    return pl.pallas_call(
        _mods_kernel,
        out_shape=jax.ShapeDtypeStruct((1, wcat.shape[1]), jnp.float32),
    )(t2d, wcat, bcat)


def _adaln_mm_kernel(x_ref, g_ref, b_ref, w_ref, wb_ref, o_ref):
    x = x_ref[...]
    mu = jnp.mean(x, axis=-1, keepdims=True)
    var = jnp.mean((x - mu) ** 2, axis=-1, keepdims=True)
    xn = (x - mu) / jnp.sqrt(var + 1e-5)
    xn = xn * g_ref[...] + b_ref[...]
    o_ref[...] = jnp.dot(xn, w_ref[...], preferred_element_type=jnp.float32) + wb_ref[...]


def _adaln_mm(x, g, b, w, wb, blk=256):
    n, d = x.shape
    dout = w.shape[1]
    return pl.pallas_call(
        _adaln_mm_kernel,
        grid=(n // blk,),
        in_specs=[
            pl.BlockSpec((blk, d), lambda i: (i, 0)),
            pl.BlockSpec((1, d), lambda i: (0, 0)),
            pl.BlockSpec((1, d), lambda i: (0, 0)),
            pl.BlockSpec((d, dout), lambda i: (0, 0)),
            pl.BlockSpec((1, dout), lambda i: (0, 0)),
        ],
        out_specs=pl.BlockSpec((blk, dout), lambda i: (i, 0)),
        out_shape=jax.ShapeDtypeStruct((n, dout), jnp.float32),
    )(x, g, b, w, wb)


def _mm_bias_kernel(x_ref, w_ref, b_ref, o_ref):
    o_ref[...] = jnp.dot(x_ref[...], w_ref[...], preferred_element_type=jnp.float32) + b_ref[...]


def _mm_bias(x, w, b):
    return pl.pallas_call(
        _mm_bias_kernel,
        out_shape=jax.ShapeDtypeStruct((x.shape[0], w.shape[1]), jnp.float32),
    )(x, w, b)


def _attn_pair_kernel(q_ref, k_ref, v_ref, o_ref):
    for h in range(2):
        sl = slice(h * HD, (h + 1) * HD)
        q = q_ref[:, sl] * (HD ** -0.5)
        k = k_ref[:, sl]
        v = v_ref[:, sl]
        s = lax.dot_general(q, k, (((1,), (1,)), ((), ())),
                            preferred_element_type=jnp.float32)
        m = jnp.max(s, axis=-1, keepdims=True)
        p = jnp.exp(s - m)
        p = p / jnp.sum(p, axis=-1, keepdims=True)
        o_ref[:, sl] = jnp.dot(p, v, preferred_element_type=jnp.float32)


def _self_attn(qkv):
    n = qkv.shape[0]
    return pl.pallas_call(
        _attn_pair_kernel,
        grid=(H // 2,),
        in_specs=[
            pl.BlockSpec((n, 128), lambda j: (0, j)),
            pl.BlockSpec((n, 128), lambda j: (0, 6 + j)),
            pl.BlockSpec((n, 128), lambda j: (0, 12 + j)),
        ],
        out_specs=pl.BlockSpec((n, 128), lambda j: (0, j)),
        out_shape=jax.ShapeDtypeStruct((n, H * HD), jnp.float32),
    )(qkv, qkv, qkv)


def _cross_attn(qh, kv):
    n = qh.shape[0]
    l = kv.shape[0]
    return pl.pallas_call(
        _attn_pair_kernel,
        grid=(H // 2,),
        in_specs=[
            pl.BlockSpec((n, 128), lambda j: (0, j)),
            pl.BlockSpec((l, 128), lambda j: (0, j)),
            pl.BlockSpec((l, 128), lambda j: (0, 6 + j)),
        ],
        out_specs=pl.BlockSpec((n, 128), lambda j: (0, j)),
        out_shape=jax.ShapeDtypeStruct((n, H * HD), jnp.float32),
    )(qh, kv, kv)


def _sa_mega_kernel(x_ref, g_ref, b_ref, wq_ref, wk_ref, wv_ref,
                    bq_ref, bk_ref, bv_ref, pw_ref, pb_ref, out_ref, xn_scr):
    j = pl.program_id(0)

    @pl.when(j == 0)
    def _():
        x = x_ref[...]
        mu = jnp.mean(x, axis=-1, keepdims=True)
        var = jnp.mean((x - mu) ** 2, axis=-1, keepdims=True)
        xn = (x - mu) / jnp.sqrt(var + 1e-5)
        xn_scr[...] = xn * g_ref[...] + b_ref[...]
        out_ref[...] = x + pb_ref[...]

    xn = xn_scr[...]
    q = jnp.dot(xn, wq_ref[...], preferred_element_type=jnp.float32) + bq_ref[...]
    k = jnp.dot(xn, wk_ref[...], preferred_element_type=jnp.float32) + bk_ref[...]
    v = jnp.dot(xn, wv_ref[...], preferred_element_type=jnp.float32) + bv_ref[...]
    n = q.shape[0]
    og = []
    for h in range(2):
        sl = slice(h * HD, (h + 1) * HD)
        kh = k[:, sl]
        vh = v[:, sl]
        o_parts = []
        for r in range(2):
            qr = q[r * (n // 2):(r + 1) * (n // 2), sl] * (HD ** -0.5)
            s = lax.dot_general(qr, kh, (((1,), (1,)), ((), ())),
                                preferred_element_type=jnp.float32)
            m = jnp.max(s, axis=-1, keepdims=True)
            p = jnp.exp(s - m)
            p = p / jnp.sum(p, axis=-1, keepdims=True)
            o_parts.append(jnp.dot(p, vh, preferred_element_type=jnp.float32))
        og.append(jnp.concatenate(o_parts, axis=0))
    ogc = jnp.concatenate(og, axis=1)
    out_ref[...] += jnp.dot(ogc, pw_ref[...], preferred_element_type=jnp.float32)


def _sa_mega(x, g, b, qkv_w, qkv_b, pw, pb):
    n, d = x.shape
    cst = lambda i: (0, 0)
    return pl.pallas_call(
        _sa_mega_kernel,
        grid=(H // 2,),
        in_specs=[
            pl.BlockSpec((n, d), cst),
            pl.BlockSpec((1, d), cst),
            pl.BlockSpec((1, d), cst),
            pl.BlockSpec((d, 128), lambda j: (0, j)),
            pl.BlockSpec((d, 128), lambda j: (0, 6 + j)),
            pl.BlockSpec((d, 128), lambda j: (0, 12 + j)),
            pl.BlockSpec((1, 128), lambda j: (0, j)),
            pl.BlockSpec((1, 128), lambda j: (0, 6 + j)),
            pl.BlockSpec((1, 128), lambda j: (0, 12 + j)),
            pl.BlockSpec((128, d), lambda j: (j, 0)),
            pl.BlockSpec((1, d), cst),
        ],
        out_specs=pl.BlockSpec((n, d), cst),
        out_shape=jax.ShapeDtypeStruct((n, d), jnp.float32),
        scratch_shapes=[pltpu.VMEM((n, d), jnp.float32)],
    )(x, g, b, qkv_w, qkv_w, qkv_w, qkv_b, qkv_b, qkv_b, pw, pb)


def _proj_res_kernel(o_ref, w_ref, b_ref, x_ref, out_ref):
    out_ref[...] = (jnp.dot(o_ref[...], w_ref[...], preferred_element_type=jnp.float32)
                    + b_ref[...] + x_ref[...])


def _proj_res(o, w, b, x, blk=256):
    n, d = o.shape
    return pl.pallas_call(
        _proj_res_kernel,
        grid=(n // blk,),
        in_specs=[
            pl.BlockSpec((blk, d), lambda i: (i, 0)),
            pl.BlockSpec((d, d), lambda i: (0, 0)),
            pl.BlockSpec((1, d), lambda i: (0, 0)),
            pl.BlockSpec((blk, d), lambda i: (i, 0)),
        ],
        out_specs=pl.BlockSpec((blk, d), lambda i: (i, 0)),
        out_shape=jax.ShapeDtypeStruct((n, d), jnp.float32),
    )(o, w, b, x)


def _gated_proj_kernel(o_ref, g_ref, w_ref, b_ref, x_ref, out_ref):
    g = jax.nn.sigmoid(g_ref[...])
    hio = lax.broadcasted_iota(jnp.int32, (H, H * HD), 0)
    cio = lax.broadcasted_iota(jnp.int32, (H, H * HD), 1)
    expand = (cio // HD == hio).astype(jnp.float32)
    ge = jnp.dot(g, expand, preferred_element_type=jnp.float32)
    og = o_ref[...] * ge
    out_ref[...] = (jnp.dot(og, w_ref[...], preferred_element_type=jnp.float32)
                    + b_ref[...] + x_ref[...])


def _gated_proj(o, g, w, b, x, blk=256):
    n, d = o.shape
    return pl.pallas_call(
        _gated_proj_kernel,
        grid=(n // blk,),
        in_specs=[
            pl.BlockSpec((blk, d), lambda i: (i, 0)),
            pl.BlockSpec((blk, H), lambda i: (i, 0)),
            pl.BlockSpec((d, d), lambda i: (0, 0)),
            pl.BlockSpec((1, d), lambda i: (0, 0)),
            pl.BlockSpec((blk, d), lambda i: (i, 0)),
        ],
        out_specs=pl.BlockSpec((blk, d), lambda i: (i, 0)),
        out_shape=jax.ShapeDtypeStruct((n, d), jnp.float32),
    )(o, g, w, b, x)


def _ca_mega_kernel(x_ref, g_ref, b_ref, wq_ref, bq_ref, gw_ref, gb_ref,
                    k_ref, v_ref, pw_ref, pb_ref, out_ref, xn_scr, gate_scr):
    j = pl.program_id(0)

    @pl.when(j == 0)
    def _():
        x = x_ref[...]
        mu = jnp.mean(x, axis=-1, keepdims=True)
        var = jnp.mean((x - mu) ** 2, axis=-1, keepdims=True)
        xn = (x - mu) / jnp.sqrt(var + 1e-5)
        xn = xn * g_ref[...] + b_ref[...]
        xn_scr[...] = xn
        gate_scr[...] = jnp.dot(xn, gw_ref[...],
                                preferred_element_type=jnp.float32) + gb_ref[...]
        out_ref[...] = x + pb_ref[...]

    xn = xn_scr[...]
    q = jnp.dot(xn, wq_ref[...], preferred_element_type=jnp.float32) + bq_ref[...]
    og = []
    for h in range(2):
        sl = slice(h * HD, (h + 1) * HD)
        qh = q[:, sl] * (HD ** -0.5)
        s = lax.dot_general(qh, k_ref[:, sl], (((1,), (1,)), ((), ())),
                            preferred_element_type=jnp.float32)
        m = jnp.max(s, axis=-1, keepdims=True)
        p = jnp.exp(s - m)
        p = p / jnp.sum(p, axis=-1, keepdims=True)
        og.append(jnp.dot(p, v_ref[:, sl], preferred_element_type=jnp.float32))
    ogc = jnp.concatenate(og, axis=1)
    hio = lax.broadcasted_iota(jnp.int32, (H, 128), 0)
    cio = lax.broadcasted_iota(jnp.int32, (H, 128), 1)
    mj = (hio == 2 * j + cio // HD).astype(jnp.float32)
    ge = jnp.dot(jax.nn.sigmoid(gate_scr[...]), mj,
                 preferred_element_type=jnp.float32)
    out_ref[...] += jnp.dot(ogc * ge, pw_ref[...],
                            preferred_element_type=jnp.float32)


def _ca_mega(x, g, b, qwm, qbm, gw, gb, kv, pw, pb):
    n, d = x.shape
    l = kv.shape[0]
    cst = lambda i: (0, 0)
    return pl.pallas_call(
        _ca_mega_kernel,
        grid=(H // 2,),
        in_specs=[
            pl.BlockSpec((n, d), cst),
            pl.BlockSpec((1, d), cst),
            pl.BlockSpec((1, d), cst),
            pl.BlockSpec((d, 128), lambda j: (0, j)),
            pl.BlockSpec((1, 128), lambda j: (0, j)),
            pl.BlockSpec((d, H), cst),
            pl.BlockSpec((1, H), cst),
            pl.BlockSpec((l, 128), lambda j: (0, j)),
            pl.BlockSpec((l, 128), lambda j: (0, 6 + j)),
            pl.BlockSpec((128, d), lambda j: (j, 0)),
            pl.BlockSpec((1, d), cst),
        ],
        out_specs=pl.BlockSpec((n, d), cst),
        out_shape=jax.ShapeDtypeStruct((n, d), jnp.float32),
        scratch_shapes=[pltpu.VMEM((n, d), jnp.float32),
                        pltpu.VMEM((n, H), jnp.float32)],
    )(x, g, b, qwm, qbm, gw, gb, kv, kv, pw, pb)


def _router_kernel(x_ref, g_ref, b_ref, rw_ref, rb_ref,
                   xn_ref, i1_ref, i2_ref, w1_ref, w2_ref):
    x = x_ref[...]
    mu = jnp.mean(x, axis=-1, keepdims=True)
    var = jnp.mean((x - mu) ** 2, axis=-1, keepdims=True)
    xn = (x - mu) / jnp.sqrt(var + 1e-5)
    xn = xn * g_ref[...] + b_ref[...]
    xn_ref[...] = xn
    logits = jnp.dot(xn, rw_ref[...], preferred_element_type=jnp.float32) + rb_ref[...]
    m = jnp.max(logits, axis=-1, keepdims=True)
    ex = jnp.exp(logits - m)
    probs = ex / jnp.sum(ex, axis=-1, keepdims=True)
    eio = lax.broadcasted_iota(jnp.int32, probs.shape, 1)
    m1 = jnp.max(probs, axis=-1, keepdims=True)
    i1 = jnp.min(jnp.where(probs == m1, eio, E), axis=-1, keepdims=True)
    p2 = jnp.where(eio == i1, -jnp.inf, probs)
    m2 = jnp.max(p2, axis=-1, keepdims=True)
    i2 = jnp.min(jnp.where(p2 == m2, eio, E), axis=-1, keepdims=True)
    s = m1 + m2
    i1_ref[...] = i1
    i2_ref[...] = i2
    w1_ref[...] = m1 / s
    w2_ref[...] = m2 / s


def _router(x, g, b, rw, rb, blk=256):
    n, d = x.shape
    return pl.pallas_call(
        _router_kernel,
        grid=(n // blk,),
        in_specs=[
            pl.BlockSpec((blk, d), lambda i: (i, 0)),
            pl.BlockSpec((1, d), lambda i: (0, 0)),
            pl.BlockSpec((1, d), lambda i: (0, 0)),
            pl.BlockSpec((d, E), lambda i: (0, 0)),
            pl.BlockSpec((1, E), lambda i: (0, 0)),
        ],
        out_specs=[
            pl.BlockSpec((blk, d), lambda i: (i, 0)),
            pl.BlockSpec((blk, 1), lambda i: (i, 0)),
            pl.BlockSpec((blk, 1), lambda i: (i, 0)),
            pl.BlockSpec((blk, 1), lambda i: (i, 0)),
            pl.BlockSpec((blk, 1), lambda i: (i, 0)),
        ],
        out_shape=[
            jax.ShapeDtypeStruct((n, d), jnp.float32),
            jax.ShapeDtypeStruct((n, 1), jnp.int32),
            jax.ShapeDtypeStruct((n, 1), jnp.int32),
            jax.ShapeDtypeStruct((n, 1), jnp.float32),
            jax.ShapeDtypeStruct((n, 1), jnp.float32),
        ],
    )(x, g, b, rw, rb)


def _pos_kernel(fi_ref, dst_ref, loc_ref, keep_ref, src_ref, carry_ref):
    j = pl.program_id(0)

    @pl.when(j == 0)
    def _():
        carry_ref[...] = jnp.zeros_like(carry_ref)

    blk = fi_ref.shape[0]
    fi = fi_ref[...]
    eio = lax.broadcasted_iota(jnp.int32, (blk, E), 1)
    oh = (fi == eio).astype(jnp.float32)
    rio = lax.broadcasted_iota(jnp.int32, (blk, blk), 0)
    cio = lax.broadcasted_iota(jnp.int32, (blk, blk), 1)
    tril = (cio <= rio).astype(jnp.float32)
    cum = jnp.dot(tril, oh, preferred_element_type=jnp.float32) + carry_ref[...]
    carry_ref[...] = cum[blk - 1:blk, :]
    pos = (jnp.sum(cum * oh, axis=-1, keepdims=True) - 1.0).astype(jnp.int32)
    keep = pos < CAP
    slot = fi * CAP + pos
    dst_ref[...] = jnp.where(keep, slot, TRASH)
    loc_ref[...] = jnp.where(keep, slot, 0)
    keep_ref[...] = keep.astype(jnp.float32)
    row = lax.broadcasted_iota(jnp.int32, (blk, 1), 0)
    src_ref[...] = (j * blk + row) // TOPK


def _positions(fi, blk=256):
    p = fi.shape[0]
    return pl.pallas_call(
        _pos_kernel,
        grid=(p // blk,),
        in_specs=[pl.BlockSpec((blk, 1), lambda i: (i, 0))],
        out_specs=[pl.BlockSpec((blk, 1), lambda i: (i, 0))] * 4,
        out_shape=[
            jax.ShapeDtypeStruct((p, 1), jnp.int32),
            jax.ShapeDtypeStruct((p, 1), jnp.int32),
            jax.ShapeDtypeStruct((p, 1), jnp.float32),
            jax.ShapeDtypeStruct((p, 1), jnp.int32),
        ],
        scratch_shapes=[pltpu.VMEM((1, E), jnp.float32)],
    )(fi)


def _ffn_kernel(d_ref, w1_ref, b1_ref, w2_ref, b2_ref, o_ref):
    d = d_ref[0]
    h = jnp.dot(d, w1_ref[0], preferred_element_type=jnp.float32) + b1_ref[0]
    h = jax.nn.gelu(h)
    o_ref[0] = jnp.dot(h, w2_ref[0], preferred_element_type=jnp.float32) + b2_ref[0]


def _ffn(dispE, w1, b1, w2, b2):
    dff = w1.shape[2]
    return pl.pallas_call(
        _ffn_kernel,
        grid=(E,),
        in_specs=[
            pl.BlockSpec((1, CAP, D), lambda e: (e, 0, 0)),
            pl.BlockSpec((1, D, dff), lambda e: (e, 0, 0)),
            pl.BlockSpec((1, 1, dff), lambda e: (e, 0, 0)),
            pl.BlockSpec((1, dff, D), lambda e: (e, 0, 0)),
            pl.BlockSpec((1, 1, D), lambda e: (e, 0, 0)),
        ],
        out_specs=pl.BlockSpec((1, CAP, D), lambda e: (e, 0, 0)),
        out_shape=jax.ShapeDtypeStruct((E, CAP, D), jnp.float32),
    )(dispE, w1, b1.reshape(E, 1, dff), w2, b2.reshape(E, 1, D))


def _moe_out_kernel(x_ref, y0_ref, y1_ref, w1_ref, w2_ref, k1_ref, k2_ref, o_ref):
    c0 = jnp.where(k1_ref[...] > 0, w1_ref[...] * y0_ref[...], 0.0)
    c1 = jnp.where(k2_ref[...] > 0, w2_ref[...] * y1_ref[...], 0.0)
    o_ref[...] = x_ref[...] + c0 + c1


def _moe_out(x, y0, y1, w1, w2, k1, k2, blk=256):
    n, d = x.shape
    row = pl.BlockSpec((blk, d), lambda i: (i, 0))
    col = pl.BlockSpec((blk, 1), lambda i: (i, 0))
    return pl.pallas_call(
        _moe_out_kernel,
        grid=(n // blk,),
        in_specs=[row, row, row, col, col, col, col],
        out_specs=row,
        out_shape=jax.ShapeDtypeStruct((n, d), jnp.float32),
    )(x, y0, y1, w1, w2, k1, k2)


_NCH = 4
_CH = PPW // _NCH


def _sc_dispatch(xn, src, dst):
    mesh = plsc.VectorSubcoreMesh(core_axis_name="c", subcore_axis_name="s")

    @functools.partial(
        pl.kernel,
        mesh=mesh,
        out_type=jax.ShapeDtypeStruct((E * CAP + 8, D), jnp.float32),
        scratch_types=[
            pltpu.VMEM((PPW,), jnp.int32),
            pltpu.VMEM((_NCH, _CH), jnp.int32),
            pltpu.VMEM((PPW, D), jnp.float32),
            pltpu.SemaphoreType.DMA,
            pltpu.SemaphoreType.DMA,
        ],
    )
    def disp_k(xn_hbm, src_hbm, dst_hbm, out_hbm, idx_s, idx_d, rows, sem_g, sem_s):
        wid = lax.axis_index("s") * 2 + lax.axis_index("c")
        base = wid * PPW
        pltpu.sync_copy(src_hbm.at[pl.ds(base, PPW)], idx_s)
        pltpu.sync_copy(dst_hbm.at[wid], idx_d)
        g = pltpu.async_copy(xn_hbm.at[idx_s.at[pl.ds(0, _CH)]],
                             rows.at[pl.ds(0, _CH)], sem_g)
        scats = []
        for i in range(_NCH):
            g.wait()
            if i + 1 < _NCH:
                g = pltpu.async_copy(
                    xn_hbm.at[idx_s.at[pl.ds((i + 1) * _CH, _CH)]],
                    rows.at[pl.ds((i + 1) * _CH, _CH)], sem_g)
            scats.append(pltpu.async_copy(rows.at[pl.ds(i * _CH, _CH)],
                                          out_hbm.at[idx_d.at[i]], sem_s))
        for s in scats:
            s.wait()

    return disp_k(xn, src, dst)


def _sc_combine(y, loc):
    mesh = plsc.VectorSubcoreMesh(core_axis_name="c", subcore_axis_name="s")
    half = PPW // 2

    @functools.partial(
        pl.kernel,
        mesh=mesh,
        out_type=jax.ShapeDtypeStruct((P_PAIRS, D), jnp.float32),
        scratch_types=[
            pltpu.VMEM((PPW,), jnp.int32),
            pltpu.VMEM((PPW, D), jnp.float32),
            pltpu.SemaphoreType.DMA,
        ],
    )
    def comb_k(y_hbm, loc_hbm, out_hbm, idx_v, rows, sem):
        wid = lax.axis_index("s") * 2 + lax.axis_index("c")
        base = wid * PPW
        pltpu.sync_copy(loc_hbm.at[pl.ds(base, PPW)], idx_v)
        g = pltpu.async_copy(y_hbm.at[idx_v.at[pl.ds(0, _CH)]],
                             rows.at[pl.ds(0, _CH)], sem)
        for i in range(_NCH):
            g.wait()
            if i + 1 < _NCH:
                g = pltpu.async_copy(
                    y_hbm.at[idx_v.at[pl.ds((i + 1) * _CH, _CH)]],
                    rows.at[pl.ds((i + 1) * _CH, _CH)], sem)
            out_off = (i // 2) * N_TOK + wid * half + (i % 2) * _CH
            pltpu.sync_copy(rows.at[pl.ds(i * _CH, _CH)],
                            out_hbm.at[pl.ds(out_off, _CH)])

    return comb_k(y, loc)


def kernel(x, c, t, params):
    B, N, Dm = x.shape
    x2d = _f32(x).reshape(N, Dm)
    c2d = _f32(c).reshape(-1, Dm)
    t2d = _f32(t)

    wcat = jnp.concatenate([params['ada1_w'], params['ada2_w'], params['ada3_w']], axis=1)
    bcat = jnp.concatenate([params['ada1_b'], params['ada2_b'], params['ada3_b']])[None, :]
    mods = _mods(t2d, wcat, bcat)
    g1, b1 = mods[:, 0:768], mods[:, 768:1536]
    g2, b2 = mods[:, 1536:2304], mods[:, 2304:3072]
    g3, b3 = mods[:, 3072:3840], mods[:, 3840:4608]

    x1 = _sa_mega(x2d, g1, b1, params['qkv_w'], params['qkv_b'][None, :],
                  params['sa_proj_w'], params['sa_proj_b'][None, :])

    qw3 = params['q_w'].reshape(Dm, H, HD + 1)
    qwm = qw3[:, :, :HD].reshape(Dm, H * HD)
    gw = qw3[:, :, HD]
    qb3 = params['q_b'].reshape(H, HD + 1)
    qbm = qb3[:, :HD].reshape(1, H * HD)
    gb = qb3[:, HD][None, :]
    kv = _mm_bias(c2d, params['kv_w'], params['kv_b'][None, :])
    x2 = _ca_mega(x1, g2, b2, qwm, qbm, gw, gb, kv,
                  params['ca_proj_w'], params['ca_proj_b'][None, :])

    xn3, i1, i2, w1n, w2n = _router(x2, g3, b3, params['r_w'], params['r_b'][None, :])
    fi = jnp.concatenate([i1, i2], axis=1).reshape(P_PAIRS, 1)
    dst, loc, keep, src = _positions(fi)

    disp = _sc_dispatch(xn3, src.reshape(P_PAIRS),
                        dst.reshape(NW, _NCH, _CH))
    dispE = disp[:E * CAP].reshape(E, CAP, Dm)
    y = _ffn(dispE, params['w1'], params['b1'], params['w2'], params['b2'])
    loc_d = loc.reshape(NW, PPW // 2, TOPK).transpose(0, 2, 1).reshape(P_PAIRS)
    ypairs = _sc_combine(y.reshape(E * CAP, Dm), loc_d)

    kp = keep.reshape(N, TOPK)
    out = _moe_out(x2, ypairs[:N_TOK], ypairs[N_TOK:], w1n, w2n,
                   kp[:, 0:1], kp[:, 1:2])
    return out.reshape(B, N, Dm)


# speedup vs baseline: 1.0131x; 1.0131x over previous
import functools

import jax
import jax.numpy as jnp
from jax import lax
from jax.experimental import pallas as pl
from jax.experimental.pallas import tpu as pltpu
from jax.experimental.pallas import tpu_sc as plsc

H = 12
HD = 64
D = 768
E = 64
CAP = 80
TOPK = 2
N_TOK = 2048
P_PAIRS = N_TOK * TOPK
NW = 32
PPW = P_PAIRS // NW
TRASH = E * CAP


def _f32(x):
    return jnp.asarray(x, jnp.float32)


def _mods_kernel(t_ref, w_ref, b_ref, o_ref):
    s = jax.nn.silu(t_ref[...])
    o_ref[...] = jnp.dot(s, w_ref[...], preferred_element_type=jnp.float32) + b_ref[...]


def _mods(t2d, wcat, bcat):
    return pl.pallas_call(
        _mods_kernel,
        out_shape=jax.ShapeDtypeStruct((1, wcat.shape[1]), jnp.float32),
    )(t2d, wcat, bcat)


def _adaln_mm_kernel(x_ref, g_ref, b_ref, w_ref, wb_ref, o_ref):
    x = x_ref[...]
    mu = jnp.mean(x, axis=-1, keepdims=True)
    var = jnp.mean((x - mu) ** 2, axis=-1, keepdims=True)
    xn = (x - mu) / jnp.sqrt(var + 1e-5)
    xn = xn * g_ref[...] + b_ref[...]
    o_ref[...] = jnp.dot(xn, w_ref[...], preferred_element_type=jnp.float32) + wb_ref[...]


def _adaln_mm(x, g, b, w, wb, blk=256):
    n, d = x.shape
    dout = w.shape[1]
    return pl.pallas_call(
        _adaln_mm_kernel,
        grid=(n // blk,),
        in_specs=[
            pl.BlockSpec((blk, d), lambda i: (i, 0)),
            pl.BlockSpec((1, d), lambda i: (0, 0)),
            pl.BlockSpec((1, d), lambda i: (0, 0)),
            pl.BlockSpec((d, dout), lambda i: (0, 0)),
            pl.BlockSpec((1, dout), lambda i: (0, 0)),
        ],
        out_specs=pl.BlockSpec((blk, dout), lambda i: (i, 0)),
        out_shape=jax.ShapeDtypeStruct((n, dout), jnp.float32),
    )(x, g, b, w, wb)


def _mm_bias_kernel(x_ref, w_ref, b_ref, o_ref):
    o_ref[...] = jnp.dot(x_ref[...], w_ref[...], preferred_element_type=jnp.float32) + b_ref[...]


def _mm_bias(x, w, b):
    return pl.pallas_call(
        _mm_bias_kernel,
        out_shape=jax.ShapeDtypeStruct((x.shape[0], w.shape[1]), jnp.float32),
    )(x, w, b)


def _attn_pair_kernel(q_ref, k_ref, v_ref, o_ref):
    for h in range(2):
        sl = slice(h * HD, (h + 1) * HD)
        q = q_ref[:, sl] * (HD ** -0.5)
        k = k_ref[:, sl]
        v = v_ref[:, sl]
        s = lax.dot_general(q, k, (((1,), (1,)), ((), ())),
                            preferred_element_type=jnp.float32)
        m = jnp.max(s, axis=-1, keepdims=True)
        p = jnp.exp(s - m)
        p = p / jnp.sum(p, axis=-1, keepdims=True)
        o_ref[:, sl] = jnp.dot(p, v, preferred_element_type=jnp.float32)


def _self_attn(qkv):
    n = qkv.shape[0]
    return pl.pallas_call(
        _attn_pair_kernel,
        grid=(H // 2,),
        in_specs=[
            pl.BlockSpec((n, 128), lambda j: (0, j)),
            pl.BlockSpec((n, 128), lambda j: (0, 6 + j)),
            pl.BlockSpec((n, 128), lambda j: (0, 12 + j)),
        ],
        out_specs=pl.BlockSpec((n, 128), lambda j: (0, j)),
        out_shape=jax.ShapeDtypeStruct((n, H * HD), jnp.float32),
    )(qkv, qkv, qkv)


def _cross_attn(qh, kv):
    n = qh.shape[0]
    l = kv.shape[0]
    return pl.pallas_call(
        _attn_pair_kernel,
        grid=(H // 2,),
        in_specs=[
            pl.BlockSpec((n, 128), lambda j: (0, j)),
            pl.BlockSpec((l, 128), lambda j: (0, j)),
            pl.BlockSpec((l, 128), lambda j: (0, 6 + j)),
        ],
        out_specs=pl.BlockSpec((n, 128), lambda j: (0, j)),
        out_shape=jax.ShapeDtypeStruct((n, H * HD), jnp.float32),
    )(qh, kv, kv)


def _sa_mega_kernel(x_ref, g_ref, b_ref, wq_ref, wk_ref, wv_ref,
                    bq_ref, bk_ref, bv_ref, pw_ref, pb_ref, out_ref, xn_scr):
    j = pl.program_id(0)

    @pl.when(j == 0)
    def _():
        x = x_ref[...]
        mu = jnp.mean(x, axis=-1, keepdims=True)
        var = jnp.mean((x - mu) ** 2, axis=-1, keepdims=True)
        xn = (x - mu) / jnp.sqrt(var + 1e-5)
        xn_scr[...] = xn * g_ref[...] + b_ref[...]
        out_ref[...] = x + pb_ref[...]

    xn = xn_scr[...]
    q = jnp.dot(xn, wq_ref[...], preferred_element_type=jnp.float32) + bq_ref[...]
    k = jnp.dot(xn, wk_ref[...], preferred_element_type=jnp.float32) + bk_ref[...]
    v = jnp.dot(xn, wv_ref[...], preferred_element_type=jnp.float32) + bv_ref[...]
    n = q.shape[0]
    og = []
    for h in range(2):
        sl = slice(h * HD, (h + 1) * HD)
        kh = k[:, sl]
        vh = v[:, sl]
        o_parts = []
        for r in range(2):
            qr = q[r * (n // 2):(r + 1) * (n // 2), sl] * (HD ** -0.5)
            s = lax.dot_general(qr, kh, (((1,), (1,)), ((), ())),
                                preferred_element_type=jnp.float32)
            m = jnp.max(s, axis=-1, keepdims=True)
            p = jnp.exp(s - m)
            p = p / jnp.sum(p, axis=-1, keepdims=True)
            o_parts.append(jnp.dot(p, vh, preferred_element_type=jnp.float32))
        og.append(jnp.concatenate(o_parts, axis=0))
    ogc = jnp.concatenate(og, axis=1)
    out_ref[...] += jnp.dot(ogc, pw_ref[...], preferred_element_type=jnp.float32)


def _sa_mega(x, g, b, qkv_w, qkv_b, pw, pb):
    n, d = x.shape
    cst = lambda i: (0, 0)
    return pl.pallas_call(
        _sa_mega_kernel,
        grid=(H // 2,),
        in_specs=[
            pl.BlockSpec((n, d), cst),
            pl.BlockSpec((1, d), cst),
            pl.BlockSpec((1, d), cst),
            pl.BlockSpec((d, 128), lambda j: (0, j)),
            pl.BlockSpec((d, 128), lambda j: (0, 6 + j)),
            pl.BlockSpec((d, 128), lambda j: (0, 12 + j)),
            pl.BlockSpec((1, 128), lambda j: (0, j)),
            pl.BlockSpec((1, 128), lambda j: (0, 6 + j)),
            pl.BlockSpec((1, 128), lambda j: (0, 12 + j)),
            pl.BlockSpec((128, d), lambda j: (j, 0)),
            pl.BlockSpec((1, d), cst),
        ],
        out_specs=pl.BlockSpec((n, d), cst),
        out_shape=jax.ShapeDtypeStruct((n, d), jnp.float32),
        scratch_shapes=[pltpu.VMEM((n, d), jnp.float32)],
    )(x, g, b, qkv_w, qkv_w, qkv_w, qkv_b, qkv_b, qkv_b, pw, pb)


def _proj_res_kernel(o_ref, w_ref, b_ref, x_ref, out_ref):
    out_ref[...] = (jnp.dot(o_ref[...], w_ref[...], preferred_element_type=jnp.float32)
                    + b_ref[...] + x_ref[...])


def _proj_res(o, w, b, x, blk=256):
    n, d = o.shape
    return pl.pallas_call(
        _proj_res_kernel,
        grid=(n // blk,),
        in_specs=[
            pl.BlockSpec((blk, d), lambda i: (i, 0)),
            pl.BlockSpec((d, d), lambda i: (0, 0)),
            pl.BlockSpec((1, d), lambda i: (0, 0)),
            pl.BlockSpec((blk, d), lambda i: (i, 0)),
        ],
        out_specs=pl.BlockSpec((blk, d), lambda i: (i, 0)),
        out_shape=jax.ShapeDtypeStruct((n, d), jnp.float32),
    )(o, w, b, x)


def _gated_proj_kernel(o_ref, g_ref, w_ref, b_ref, x_ref, out_ref):
    g = jax.nn.sigmoid(g_ref[...])
    hio = lax.broadcasted_iota(jnp.int32, (H, H * HD), 0)
    cio = lax.broadcasted_iota(jnp.int32, (H, H * HD), 1)
    expand = (cio // HD == hio).astype(jnp.float32)
    ge = jnp.dot(g, expand, preferred_element_type=jnp.float32)
    og = o_ref[...] * ge
    out_ref[...] = (jnp.dot(og, w_ref[...], preferred_element_type=jnp.float32)
                    + b_ref[...] + x_ref[...])


def _gated_proj(o, g, w, b, x, blk=256):
    n, d = o.shape
    return pl.pallas_call(
        _gated_proj_kernel,
        grid=(n // blk,),
        in_specs=[
            pl.BlockSpec((blk, d), lambda i: (i, 0)),
            pl.BlockSpec((blk, H), lambda i: (i, 0)),
            pl.BlockSpec((d, d), lambda i: (0, 0)),
            pl.BlockSpec((1, d), lambda i: (0, 0)),
            pl.BlockSpec((blk, d), lambda i: (i, 0)),
        ],
        out_specs=pl.BlockSpec((blk, d), lambda i: (i, 0)),
        out_shape=jax.ShapeDtypeStruct((n, d), jnp.float32),
    )(o, g, w, b, x)


def _router_kernel(x_ref, g_ref, b_ref, rw_ref, rb_ref,
                   xn_ref, i1_ref, i2_ref, w1_ref, w2_ref):
    x = x_ref[...]
    mu = jnp.mean(x, axis=-1, keepdims=True)
    var = jnp.mean((x - mu) ** 2, axis=-1, keepdims=True)
    xn = (x - mu) / jnp.sqrt(var + 1e-5)
    xn = xn * g_ref[...] + b_ref[...]
    xn_ref[...] = xn
    logits = jnp.dot(xn, rw_ref[...], preferred_element_type=jnp.float32) + rb_ref[...]
    m = jnp.max(logits, axis=-1, keepdims=True)
    ex = jnp.exp(logits - m)
    probs = ex / jnp.sum(ex, axis=-1, keepdims=True)
    eio = lax.broadcasted_iota(jnp.int32, probs.shape, 1)
    m1 = jnp.max(probs, axis=-1, keepdims=True)
    i1 = jnp.min(jnp.where(probs == m1, eio, E), axis=-1, keepdims=True)
    p2 = jnp.where(eio == i1, -jnp.inf, probs)
    m2 = jnp.max(p2, axis=-1, keepdims=True)
    i2 = jnp.min(jnp.where(p2 == m2, eio, E), axis=-1, keepdims=True)
    s = m1 + m2
    i1_ref[...] = i1
    i2_ref[...] = i2
    w1_ref[...] = m1 / s
    w2_ref[...] = m2 / s


def _router(x, g, b, rw, rb, blk=256):
    n, d = x.shape
    return pl.pallas_call(
        _router_kernel,
        grid=(n // blk,),
        in_specs=[
            pl.BlockSpec((blk, d), lambda i: (i, 0)),
            pl.BlockSpec((1, d), lambda i: (0, 0)),
            pl.BlockSpec((1, d), lambda i: (0, 0)),
            pl.BlockSpec((d, E), lambda i: (0, 0)),
            pl.BlockSpec((1, E), lambda i: (0, 0)),
        ],
        out_specs=[
            pl.BlockSpec((blk, d), lambda i: (i, 0)),
            pl.BlockSpec((blk, 1), lambda i: (i, 0)),
            pl.BlockSpec((blk, 1), lambda i: (i, 0)),
            pl.BlockSpec((blk, 1), lambda i: (i, 0)),
            pl.BlockSpec((blk, 1), lambda i: (i, 0)),
        ],
        out_shape=[
            jax.ShapeDtypeStruct((n, d), jnp.float32),
            jax.ShapeDtypeStruct((n, 1), jnp.int32),
            jax.ShapeDtypeStruct((n, 1), jnp.int32),
            jax.ShapeDtypeStruct((n, 1), jnp.float32),
            jax.ShapeDtypeStruct((n, 1), jnp.float32),
        ],
    )(x, g, b, rw, rb)


def _pos_kernel(fi_ref, dst_ref, loc_ref, keep_ref, src_ref, carry_ref):
    j = pl.program_id(0)

    @pl.when(j == 0)
    def _():
        carry_ref[...] = jnp.zeros_like(carry_ref)

    blk = fi_ref.shape[0]
    fi = fi_ref[...]
    eio = lax.broadcasted_iota(jnp.int32, (blk, E), 1)
    oh = (fi == eio).astype(jnp.float32)
    rio = lax.broadcasted_iota(jnp.int32, (blk, blk), 0)
    cio = lax.broadcasted_iota(jnp.int32, (blk, blk), 1)
    tril = (cio <= rio).astype(jnp.float32)
    cum = jnp.dot(tril, oh, preferred_element_type=jnp.float32) + carry_ref[...]
    carry_ref[...] = cum[blk - 1:blk, :]
    pos = (jnp.sum(cum * oh, axis=-1, keepdims=True) - 1.0).astype(jnp.int32)
    keep = pos < CAP
    slot = fi * CAP + pos
    dst_ref[...] = jnp.where(keep, slot, TRASH)
    loc_ref[...] = jnp.where(keep, slot, 0)
    keep_ref[...] = keep.astype(jnp.float32)
    row = lax.broadcasted_iota(jnp.int32, (blk, 1), 0)
    src_ref[...] = (j * blk + row) // TOPK


def _positions(fi, blk=256):
    p = fi.shape[0]
    return pl.pallas_call(
        _pos_kernel,
        grid=(p // blk,),
        in_specs=[pl.BlockSpec((blk, 1), lambda i: (i, 0))],
        out_specs=[pl.BlockSpec((blk, 1), lambda i: (i, 0))] * 4,
        out_shape=[
            jax.ShapeDtypeStruct((p, 1), jnp.int32),
            jax.ShapeDtypeStruct((p, 1), jnp.int32),
            jax.ShapeDtypeStruct((p, 1), jnp.float32),
            jax.ShapeDtypeStruct((p, 1), jnp.int32),
        ],
        scratch_shapes=[pltpu.VMEM((1, E), jnp.float32)],
    )(fi)


def _ffn_kernel(d_ref, w1_ref, b1_ref, w2_ref, b2_ref, o_ref):
    d = d_ref[0]
    h = jnp.dot(d, w1_ref[0], preferred_element_type=jnp.float32) + b1_ref[0]
    h = jax.nn.gelu(h)
    o_ref[0] = jnp.dot(h, w2_ref[0], preferred_element_type=jnp.float32) + b2_ref[0]


def _ffn(dispE, w1, b1, w2, b2):
    dff = w1.shape[2]
    return pl.pallas_call(
        _ffn_kernel,
        grid=(E,),
        in_specs=[
            pl.BlockSpec((1, CAP, D), lambda e: (e, 0, 0)),
            pl.BlockSpec((1, D, dff), lambda e: (e, 0, 0)),
            pl.BlockSpec((1, 1, dff), lambda e: (e, 0, 0)),
            pl.BlockSpec((1, dff, D), lambda e: (e, 0, 0)),
            pl.BlockSpec((1, 1, D), lambda e: (e, 0, 0)),
        ],
        out_specs=pl.BlockSpec((1, CAP, D), lambda e: (e, 0, 0)),
        out_shape=jax.ShapeDtypeStruct((E, CAP, D), jnp.float32),
    )(dispE, w1, b1.reshape(E, 1, dff), w2, b2.reshape(E, 1, D))


def _moe_out_kernel(x_ref, y0_ref, y1_ref, w1_ref, w2_ref, k1_ref, k2_ref, o_ref):
    c0 = jnp.where(k1_ref[...] > 0, w1_ref[...] * y0_ref[...], 0.0)
    c1 = jnp.where(k2_ref[...] > 0, w2_ref[...] * y1_ref[...], 0.0)
    o_ref[...] = x_ref[...] + c0 + c1


def _moe_out(x, y0, y1, w1, w2, k1, k2, blk=256):
    n, d = x.shape
    row = pl.BlockSpec((blk, d), lambda i: (i, 0))
    col = pl.BlockSpec((blk, 1), lambda i: (i, 0))
    return pl.pallas_call(
        _moe_out_kernel,
        grid=(n // blk,),
        in_specs=[row, row, row, col, col, col, col],
        out_specs=row,
        out_shape=jax.ShapeDtypeStruct((n, d), jnp.float32),
    )(x, y0, y1, w1, w2, k1, k2)


_NCH = 4
_CH = PPW // _NCH


def _sc_dispatch(xn, src, dst):
    mesh = plsc.VectorSubcoreMesh(core_axis_name="c", subcore_axis_name="s")

    @functools.partial(
        pl.kernel,
        mesh=mesh,
        out_type=jax.ShapeDtypeStruct((E * CAP + 8, D), jnp.float32),
        scratch_types=[
            pltpu.VMEM((PPW,), jnp.int32),
            pltpu.VMEM((_NCH, _CH), jnp.int32),
            pltpu.VMEM((PPW, D), jnp.float32),
            pltpu.SemaphoreType.DMA,
            pltpu.SemaphoreType.DMA,
        ],
    )
    def disp_k(xn_hbm, src_hbm, dst_hbm, out_hbm, idx_s, idx_d, rows, sem_g, sem_s):
        wid = lax.axis_index("s") * 2 + lax.axis_index("c")
        base = wid * PPW
        pltpu.sync_copy(src_hbm.at[pl.ds(base, PPW)], idx_s)
        pltpu.sync_copy(dst_hbm.at[wid], idx_d)
        g = pltpu.async_copy(xn_hbm.at[idx_s.at[pl.ds(0, _CH)]],
                             rows.at[pl.ds(0, _CH)], sem_g)
        scats = []
        for i in range(_NCH):
            g.wait()
            if i + 1 < _NCH:
                g = pltpu.async_copy(
                    xn_hbm.at[idx_s.at[pl.ds((i + 1) * _CH, _CH)]],
                    rows.at[pl.ds((i + 1) * _CH, _CH)], sem_g)
            scats.append(pltpu.async_copy(rows.at[pl.ds(i * _CH, _CH)],
                                          out_hbm.at[idx_d.at[i]], sem_s))
        for s in scats:
            s.wait()

    return disp_k(xn, src, dst)


def _sc_combine(y, loc):
    mesh = plsc.VectorSubcoreMesh(core_axis_name="c", subcore_axis_name="s")
    half = PPW // 2

    @functools.partial(
        pl.kernel,
        mesh=mesh,
        out_type=jax.ShapeDtypeStruct((P_PAIRS, D), jnp.float32),
        scratch_types=[
            pltpu.VMEM((PPW,), jnp.int32),
            pltpu.VMEM((PPW, D), jnp.float32),
            pltpu.SemaphoreType.DMA,
        ],
    )
    def comb_k(y_hbm, loc_hbm, out_hbm, idx_v, rows, sem):
        wid = lax.axis_index("s") * 2 + lax.axis_index("c")
        base = wid * PPW
        pltpu.sync_copy(loc_hbm.at[pl.ds(base, PPW)], idx_v)
        g = pltpu.async_copy(y_hbm.at[idx_v.at[pl.ds(0, _CH)]],
                             rows.at[pl.ds(0, _CH)], sem)
        for i in range(_NCH):
            g.wait()
            if i + 1 < _NCH:
                g = pltpu.async_copy(
                    y_hbm.at[idx_v.at[pl.ds((i + 1) * _CH, _CH)]],
                    rows.at[pl.ds((i + 1) * _CH, _CH)], sem)
            out_off = (i // 2) * N_TOK + wid * half + (i % 2) * _CH
            pltpu.sync_copy(rows.at[pl.ds(i * _CH, _CH)],
                            out_hbm.at[pl.ds(out_off, _CH)])

    return comb_k(y, loc)


def kernel(x, c, t, params):
    B, N, Dm = x.shape
    x2d = _f32(x).reshape(N, Dm)
    c2d = _f32(c).reshape(-1, Dm)
    t2d = _f32(t)

    wcat = jnp.concatenate([params['ada1_w'], params['ada2_w'], params['ada3_w']], axis=1)
    bcat = jnp.concatenate([params['ada1_b'], params['ada2_b'], params['ada3_b']])[None, :]
    mods = _mods(t2d, wcat, bcat)
    g1, b1 = mods[:, 0:768], mods[:, 768:1536]
    g2, b2 = mods[:, 1536:2304], mods[:, 2304:3072]
    g3, b3 = mods[:, 3072:3840], mods[:, 3840:4608]

    x1 = _sa_mega(x2d, g1, b1, params['qkv_w'], params['qkv_b'][None, :],
                  params['sa_proj_w'], params['sa_proj_b'][None, :])

    qw3 = params['q_w'].reshape(Dm, H, HD + 1)
    qwm = qw3[:, :, :HD].reshape(Dm, H * HD)
    gw = qw3[:, :, HD]
    qb3 = params['q_b'].reshape(H, HD + 1)
    qbm = qb3[:, :HD].reshape(1, H * HD)
    gb = qb3[:, HD][None, :]
    wq = jnp.concatenate([qwm, gw], axis=1)
    wqb = jnp.concatenate([qbm, gb], axis=1)
    qo = _adaln_mm(x1, g2, b2, wq, wqb)
    qh = qo[:, :H * HD]
    gate = qo[:, H * HD:]
    kv = _mm_bias(c2d, params['kv_w'], params['kv_b'][None, :])
    o_ca = _cross_attn(qh, kv)
    x2 = _gated_proj(o_ca, gate, params['ca_proj_w'], params['ca_proj_b'][None, :], x1)

    xn3, i1, i2, w1n, w2n = _router(x2, g3, b3, params['r_w'], params['r_b'][None, :])
    fi = jnp.concatenate([i1, i2], axis=1).reshape(P_PAIRS, 1)
    dst, loc, keep, src = _positions(fi)

    disp = _sc_dispatch(xn3, src.reshape(P_PAIRS),
                        dst.reshape(NW, _NCH, _CH))
    dispE = disp[:E * CAP].reshape(E, CAP, Dm)
    y = _ffn(dispE, params['w1'], params['b1'], params['w2'], params['b2'])
    loc_d = loc.reshape(NW, PPW // 2, TOPK).transpose(0, 2, 1).reshape(P_PAIRS)
    ypairs = _sc_combine(y.reshape(E * CAP, Dm), loc_d)

    kp = keep.reshape(N, TOPK)
    out = _moe_out(x2, ypairs[:N_TOK], ypairs[N_TOK:], w1n, w2n,
                   kp[:, 0:1], kp[:, 1:2])
    return out.reshape(B, N, Dm)
